# baseline (speedup 1.0000x reference)
import functools

import jax
import jax.numpy as jnp
from jax import lax
from jax.experimental import pallas as pl
from jax.experimental.pallas import tpu as pltpu
from jax.experimental.pallas import tpu_sc as plsc

N_TYPES = 1000
TAB_PAD = 1024
BATCH = 16384
NC, NS, L = 1, 16, 16
NW = NC * NS
B_PER_W = BATCH // NW
SLICE = TAB_PAD // NW

_LN2 = 0.6931471805599453
_SQRT2 = 1.4142135623730951


def _softlog(x):
    ib = lax.bitcast_convert_type(x, jnp.int32)
    e = ((ib >> 23) & 0xFF) - 127
    m = lax.bitcast_convert_type((ib & 0x007FFFFF) | 0x3F800000, jnp.float32)
    big = m > _SQRT2
    m = jnp.where(big, m * 0.5, m)
    e = jnp.where(big, e + 1, e)
    s = (m - 1.0) / (m + 1.0)
    z = s * s
    p = 1.0 / 9.0
    p = p * z + 1.0 / 7.0
    p = p * z + 1.0 / 5.0
    p = p * z + 1.0 / 3.0
    p = p * z + 1.0
    return e.astype(jnp.float32) * _LN2 + 2.0 * s * p


@functools.partial(
    pl.kernel,
    mesh=plsc.VectorSubcoreMesh(
        core_axis_name="c", subcore_axis_name="s", num_cores=NC
    ),
    out_type=jax.ShapeDtypeStruct((BATCH,), jnp.float32),
    scratch_types=[
        pltpu.VMEM((TAB_PAD,), jnp.float32),
        pltpu.VMEM((B_PER_W,), jnp.int32),
        pltpu.VMEM((B_PER_W,), jnp.float32),
        pltpu.VMEM_SHARED((TAB_PAD,), jnp.float32),
        pltpu.SemaphoreType.DMA,
        pltpu.SemaphoreType.DMA,
    ],
    compiler_params=pltpu.CompilerParams(needs_layout_passes=False),
)
def _sc_lookup_log(
    tab_hbm, idx_hbm, out_hbm, tab_v, idx_v, out_v, tab_sh, sem_t, sem_i
):
    wid = lax.axis_index("s") * NC + lax.axis_index("c")
    base = wid * B_PER_W
    cp_t = pltpu.async_copy(tab_hbm, tab_v.at[pl.ds(0, N_TYPES)], sem_t)
    cp_i = pltpu.async_copy(idx_hbm.at[pl.ds(base, B_PER_W)], idx_v, sem_i)
    cp_t.wait()

    sbase = wid * SLICE
    for j in range(SLICE // L):
        off = sbase + j * L
        tab_v[pl.ds(off, L)] = _softlog(tab_v[pl.ds(off, L)])

    pltpu.sync_copy(tab_v.at[pl.ds(sbase, SLICE)], tab_sh.at[pl.ds(sbase, SLICE)])
    plsc.subcore_barrier()
    pltpu.sync_copy(tab_sh, tab_v)
    cp_i.wait()

    def step(i, carry):
        idx = idx_v[pl.ds(i * L, L)]
        out_v[pl.ds(i * L, L)] = plsc.load_gather(tab_v, [idx])
        return carry

    lax.fori_loop(0, B_PER_W // L, step, 0)
    pltpu.sync_copy(out_v, out_hbm.at[pl.ds(base, B_PER_W)])


def kernel(probabilities, c):
    return _sc_lookup_log(probabilities, c.astype(jnp.int32))


# speedup vs baseline: 1.0322x; 1.0322x over previous
import functools

import jax
import jax.numpy as jnp
from jax import lax
from jax.experimental import pallas as pl
from jax.experimental.pallas import tpu as pltpu
from jax.experimental.pallas import tpu_sc as plsc

N_TYPES = 1000
BATCH = 16384
NC, NS, L = 1, 16, 16
NW = NC * NS
B_PER_W = BATCH // NW


def _log_body(p_ref, o_ref):
    o_ref[...] = jnp.log(p_ref[...])


@functools.partial(
    pl.kernel,
    mesh=plsc.VectorSubcoreMesh(
        core_axis_name="c", subcore_axis_name="s", num_cores=1
    ),
    out_type=jax.ShapeDtypeStruct((BATCH,), jnp.float32),
    scratch_types=[
        pltpu.VMEM((N_TYPES,), jnp.float32),
        pltpu.VMEM((B_PER_W,), jnp.int32),
        pltpu.VMEM((B_PER_W,), jnp.float32),
        pltpu.SemaphoreType.DMA,
        pltpu.SemaphoreType.DMA,
    ],
    compiler_params=pltpu.CompilerParams(needs_layout_passes=False),
)
def _sc_gather(tab_hbm, idx_hbm, out_hbm, tab_v, idx_v, out_v, sem_t, sem_i):
    wid = lax.axis_index("s") * NC + lax.axis_index("c")
    base = wid * B_PER_W
    cp_t = pltpu.async_copy(tab_hbm, tab_v, sem_t)
    cp_i = pltpu.async_copy(idx_hbm.at[pl.ds(base, B_PER_W)], idx_v, sem_i)
    cp_t.wait()
    cp_i.wait()

    @plsc.parallel_loop(0, B_PER_W // L, step=1)
    def step(i):
        idx = idx_v[pl.ds(i * L, L)]
        out_v[pl.ds(i * L, L)] = plsc.load_gather(tab_v, [idx])
    pltpu.sync_copy(out_v, out_hbm.at[pl.ds(base, B_PER_W)])


def kernel(probabilities, c):
    log_tab = pl.pallas_call(
        _log_body,
        out_shape=jax.ShapeDtypeStruct((N_TYPES,), jnp.float32),
    )(probabilities)
    return _sc_gather(log_tab, c.astype(jnp.int32))


# speedup vs baseline: 1.0365x; 1.0042x over previous
import functools

import jax
import jax.numpy as jnp
from jax import lax
from jax.experimental import pallas as pl
from jax.experimental.pallas import tpu as pltpu
from jax.experimental.pallas import tpu_sc as plsc

N_TYPES = 1000
BATCH = 16384
NC, NS, L = 1, 16, 16
NW = NC * NS
B_PER_W = BATCH // NW


def _log_body(p_ref, o_ref):
    o_ref[...] = jnp.log(p_ref[...])


@functools.partial(
    pl.kernel,
    mesh=plsc.VectorSubcoreMesh(
        core_axis_name="c", subcore_axis_name="s", num_cores=1
    ),
    out_type=jax.ShapeDtypeStruct((BATCH,), jnp.float32),
    scratch_types=[
        pltpu.VMEM((N_TYPES,), jnp.float32),
        pltpu.VMEM((B_PER_W,), jnp.int32),
        pltpu.VMEM((B_PER_W,), jnp.float32),
        pltpu.SemaphoreType.DMA,
        pltpu.SemaphoreType.DMA,
    ],
    compiler_params=pltpu.CompilerParams(needs_layout_passes=False),
)
def _sc_gather(tab_hbm, idx_hbm, out_hbm, tab_v, idx_v, out_v, sem_t, sem_i):
    wid = lax.axis_index("s") * NC + lax.axis_index("c")
    base = wid * B_PER_W
    cp_t = pltpu.async_copy(tab_hbm, tab_v, sem_t)
    cp_i = pltpu.async_copy(idx_hbm.at[pl.ds(base, B_PER_W)], idx_v, sem_i)
    cp_t.wait()
    cp_i.wait()

    def step(i, carry):
        idx = idx_v[pl.ds(i * L, L)]
        out_v[pl.ds(i * L, L)] = plsc.load_gather(tab_v, [idx])
        return carry

    lax.fori_loop(0, B_PER_W // L, step, 0)
    pltpu.sync_copy(out_v, out_hbm.at[pl.ds(base, B_PER_W)])


def kernel(probabilities, c):
    log_tab = pl.pallas_call(
        _log_body,
        out_shape=jax.ShapeDtypeStruct((N_TYPES,), jnp.float32),
    )(probabilities)
    return _sc_gather(log_tab, c.astype(jnp.int32))
